# Initial kernel scaffold; baseline (speedup 1.0000x reference)
import jax
import jax.numpy as jnp
from jax.experimental import pallas as pl


def kernel(q, k, v):
    raise NotImplementedError("write your pallas kernel here")


# speedup vs baseline: 3.9182x; 3.9182x over previous
import math

import jax
import jax.numpy as jnp
from jax.experimental import pallas as pl
from jax.experimental.pallas import tpu as pltpu

_Q = 2048
_KV = 2048
_D = 1024
_B = 2
_C = 5
_U_ACT = int(_C * math.log(_Q))
_U_SAMP = int(_Q * math.log(_KV))
_PAD = 64
_QCHUNK = 256
_NQ = _Q // _QCHUNK
_NEG_INF = float("-inf")


def _m_kernel(w_ref, bias_ref, q_ref, k_ref, m_ref):
    s = jax.lax.dot_general(q_ref[0], k_ref[0], (((1,), (1,)), ((), ())),
                            preferred_element_type=jnp.float32)
    mx = jnp.max(s + bias_ref[0], axis=1)
    mean = jnp.sum(s * w_ref[0], axis=1)
    m_ref[0, 0, :] = mx - mean


def _select_attend_kernel(m_ref, q_ref, k_ref, v_ref,
                          s1_ref, idx_ref, vmean_ref):
    M2 = m_ref[0, 0, :].reshape(16, 128)
    flat = (jax.lax.broadcasted_iota(jnp.int32, (16, 128), 0) * 128
            + jax.lax.broadcasted_iota(jnp.int32, (16, 128), 1))
    slot_col = jax.lax.broadcasted_iota(jnp.int32, (_PAD, 1), 0)
    slot_row = jax.lax.broadcasted_iota(jnp.int32, (1, _PAD), 1)

    def body(i, carry):
        m2, idx_col, idx_row = carry
        mx = jnp.max(m2)
        idx = jnp.min(jnp.where(m2 == mx, flat, jnp.int32(_Q)))
        idx_col = jnp.where(slot_col == i, idx, idx_col)
        idx_row = jnp.where(slot_row == i, idx, idx_row)
        m2 = jnp.where(flat == idx, _NEG_INF, m2)
        return m2, idx_col, idx_row

    idx_col0 = jnp.full((_PAD, 1), -1, jnp.int32)
    idx_row0 = jnp.full((1, _PAD), -1, jnp.int32)
    _, idx_col, idx_row = jax.lax.fori_loop(
        0, _U_ACT, body, (M2, idx_col0, idx_row0))
    idx_ref[0, 0, :] = idx_row[0, :]

    onehot = (jax.lax.broadcasted_iota(jnp.int32, (_PAD, _Q), 1)
              == idx_col).astype(jnp.float32)
    q_bar = jax.lax.dot_general(onehot, q_ref[0], (((1,), (0,)), ((), ())),
                                preferred_element_type=jnp.float32)

    scale = 1.0 / math.sqrt(_KV)
    att = jax.lax.dot_general(q_bar, k_ref[0], (((1,), (1,)), ((), ())),
                              preferred_element_type=jnp.float32) * scale
    att = att - jnp.max(att, axis=1, keepdims=True)
    att = jnp.exp(att)
    att = att / jnp.sum(att, axis=1, keepdims=True)
    s1_ref[0] = jax.lax.dot_general(att, v_ref[0], (((1,), (0,)), ((), ())),
                                    preferred_element_type=jnp.float32)
    vmean_ref[0, 0, :] = jnp.mean(v_ref[0], axis=0)


def _output_kernel(s1_ref, idx_ref, vmean_ref, out_ref):
    c = pl.program_id(1)
    onehot_t = ((jax.lax.broadcasted_iota(jnp.int32, (_QCHUNK, _PAD), 0)
                 + c * _QCHUNK) == idx_ref[0]).astype(jnp.float32)
    scattered = jax.lax.dot_general(
        onehot_t, s1_ref[0], (((1,), (0,)), ((), ())),
        preferred_element_type=jnp.float32)
    unsel = 1.0 - jnp.sum(onehot_t, axis=1, keepdims=True)
    out_ref[0] = scattered + unsel * vmean_ref[0]


def kernel(q, k, v):
    idx = jax.random.randint(jax.random.key(42), (_B, _U_SAMP), 0, _KV)
    counts = jax.vmap(
        lambda ix: jnp.zeros((_KV,), jnp.float32).at[ix].add(1.0))(idx)
    w = (counts / _U_SAMP).reshape(_B, 1, _KV)
    bias = jnp.where(counts > 0, 0.0, _NEG_INF).astype(jnp.float32)
    bias = bias.reshape(_B, 1, _KV)

    M = pl.pallas_call(
        _m_kernel,
        grid=(_B, _NQ),
        in_specs=[
            pl.BlockSpec((1, 1, _KV), lambda b, i: (b, 0, 0)),
            pl.BlockSpec((1, 1, _KV), lambda b, i: (b, 0, 0)),
            pl.BlockSpec((1, _QCHUNK, _D), lambda b, i: (b, i, 0)),
            pl.BlockSpec((1, _KV, _D), lambda b, i: (b, 0, 0)),
        ],
        out_specs=pl.BlockSpec((1, 1, _QCHUNK), lambda b, i: (b, 0, i)),
        out_shape=jax.ShapeDtypeStruct((_B, 1, _Q), jnp.float32),
        compiler_params=pltpu.CompilerParams(
            dimension_semantics=("arbitrary", "arbitrary")),
    )(w, bias, q, k)

    s1, top_idx, v_mean = pl.pallas_call(
        _select_attend_kernel,
        grid=(_B,),
        in_specs=[
            pl.BlockSpec((1, 1, _Q), lambda b: (b, 0, 0)),
            pl.BlockSpec((1, _Q, _D), lambda b: (b, 0, 0)),
            pl.BlockSpec((1, _KV, _D), lambda b: (b, 0, 0)),
            pl.BlockSpec((1, _KV, _D), lambda b: (b, 0, 0)),
        ],
        out_specs=[
            pl.BlockSpec((1, _PAD, _D), lambda b: (b, 0, 0)),
            pl.BlockSpec((1, 1, _PAD), lambda b: (b, 0, 0)),
            pl.BlockSpec((1, 1, _D), lambda b: (b, 0, 0)),
        ],
        out_shape=[
            jax.ShapeDtypeStruct((_B, _PAD, _D), jnp.float32),
            jax.ShapeDtypeStruct((_B, 1, _PAD), jnp.int32),
            jax.ShapeDtypeStruct((_B, 1, _D), jnp.float32),
        ],
        compiler_params=pltpu.CompilerParams(
            dimension_semantics=("arbitrary",)),
    )(M, q, k, v)

    return pl.pallas_call(
        _output_kernel,
        grid=(_B, _NQ),
        in_specs=[
            pl.BlockSpec((1, _PAD, _D), lambda b, i: (b, 0, 0)),
            pl.BlockSpec((1, 1, _PAD), lambda b, i: (b, 0, 0)),
            pl.BlockSpec((1, 1, _D), lambda b, i: (b, 0, 0)),
        ],
        out_specs=pl.BlockSpec((1, _QCHUNK, _D), lambda b, i: (b, i, 0)),
        out_shape=jax.ShapeDtypeStruct((_B, _Q, _D), jnp.float32),
        compiler_params=pltpu.CompilerParams(
            dimension_semantics=("arbitrary", "arbitrary")),
    )(s1, top_idx, v_mean)
